# baseline (speedup 1.0000x reference)
import functools

import jax
import jax.numpy as jnp
from jax import lax
from jax.experimental import pallas as pl
from jax.experimental.pallas import tpu as pltpu
from jax.experimental.pallas import tpu_sc as plsc

N_SONGS = 100000
N_PATTERNS = 100
BATCH = 16384

NC, NS, L = 2, 16, 16
NW = NC * NS

NG = 12
RG = 8
NROW_TAIL = N_PATTERNS - NG * RG
T_COMMON = 13
NEXTRA = 21
NT_SC = T_COMMON * NW + NEXTRA
C0 = NT_SC * 128
W_COMMON = T_COMMON * 128
W_EXTRA = 128
W = W_COMMON + W_EXTRA
NCH = W // L
NCH_COMMON = W_COMMON // L
PD = 128
BPW = BATCH // NW

CB = 128 * 23
NB0 = C0 // CB
NBLK = 15

BB = 1024

_mesh = plsc.VectorSubcoreMesh(core_axis_name="c", subcore_axis_name="s",
                               num_cores=NC, num_subcores=NS)

_GATHER_DNUMS = lax.GatherDimensionNumbers(
    offset_dims=(), collapsed_slice_dims=(0,), start_index_map=(0,))


def _permute(v, perm):
    return lax.gather(v, perm[:, None], _GATHER_DNUMS, slice_sizes=(1,),
                      mode=lax.GatherScatterMode.PROMISE_IN_BOUNDS)


def _lane_sum(v):
    idx = lax.iota(jnp.int32, L)
    for sh in (8, 4, 2, 1):
        v = v + _permute(v, jnp.bitwise_xor(idx, sh))
    return v


@functools.partial(
    pl.kernel,
    out_type=(
        jax.ShapeDtypeStruct((NW * PD,), jnp.float32),
        jax.ShapeDtypeStruct((BATCH,), jnp.float32),
    ),
    mesh=_mesh,
    compiler_params=pltpu.CompilerParams(needs_layout_passes=False,
                                         skip_device_barrier=True),
    scratch_types=[
        pltpu.VMEM((W,), jnp.float32),
        pltpu.VMEM((3 * RG, W), jnp.float32),
        pltpu.VMEM((PD,), jnp.float32),
        pltpu.VMEM((BPW,), jnp.int32),
        pltpu.VMEM((BPW,), jnp.float32),
        pltpu.SemaphoreType.DMA,
        pltpu.SemaphoreType.DMA,
        pltpu.SemaphoreType.DMA,
        pltpu.SemaphoreType.DMA,
    ],
)
def _scpart(mat_hbm, tbl_hbm, title_hbm, part_hbm, pref_hbm,
            tbl_v, buf_v, den_v, idx_v, pref_v, sem0, sem1, sem2, gsem):
    cid = lax.axis_index("c")
    wid = lax.axis_index("s") * NC + cid
    zeros = jnp.zeros((L,), jnp.float32)
    sems = (sem0, sem1, sem2)
    has_extra = wid < NEXTRA
    base = 128 * (T_COMMON * wid + jnp.minimum(wid, NEXTRA))

    def issue(g, b):
        pltpu.async_copy(
            mat_hbm.at[pl.ds(g * RG, RG), pl.ds(base, W_COMMON)],
            buf_v.at[pl.ds(b * RG, RG), pl.ds(0, W_COMMON)], sems[b])

        @pl.when(has_extra)
        def _():
            pltpu.async_copy(
                mat_hbm.at[pl.ds(g * RG, RG), pl.ds(base + W_COMMON,
                                                    W_EXTRA)],
                buf_v.at[pl.ds(b * RG, RG), pl.ds(W_COMMON, W_EXTRA)],
                sems[b])

    def wait(b):
        pltpu.make_async_copy(
            mat_hbm.at[pl.ds(0, RG), pl.ds(0, W_COMMON)],
            buf_v.at[pl.ds(b * RG, RG), pl.ds(0, W_COMMON)], sems[b]).wait()

        @pl.when(has_extra)
        def _():
            pltpu.make_async_copy(
                mat_hbm.at[pl.ds(0, RG), pl.ds(0, W_EXTRA)],
                buf_v.at[pl.ds(b * RG, RG), pl.ds(W_COMMON, W_EXTRA)],
                sems[b]).wait()

    issue(0, 0)
    issue(1, 1)
    issue(2, 2)
    pltpu.sync_copy(tbl_hbm.at[pl.ds(base, W_COMMON)],
                    tbl_v.at[pl.ds(0, W_COMMON)])

    @pl.when(has_extra)
    def _():
        pltpu.sync_copy(tbl_hbm.at[pl.ds(base + W_COMMON, W_EXTRA)],
                        tbl_v.at[pl.ds(W_COMMON, W_EXTRA)])

    bout = wid * BPW
    pltpu.sync_copy(title_hbm.at[pl.ds(bout, BPW)], idx_v)
    gather = pltpu.async_copy(tbl_hbm.at[idx_v], pref_v, gsem)

    for c in range(PD // L):
        den_v[pl.ds(c * L, L)] = zeros

    @pl.when(jnp.logical_not(has_extra))
    def _():
        for c in range(NCH_COMMON, NCH):
            tbl_v[pl.ds(c * L, L)] = zeros
            for r in range(3 * RG):
                buf_v[r, pl.ds(c * L, L)] = zeros

    lane0 = lax.iota(jnp.int32, L) == 0

    def compute_and_store(p0, b):
        def chunk_body(j, accs):
            col = j * L
            t = tbl_v[pl.ds(col, L)]
            return tuple(accs[r] + buf_v[b * RG + r, pl.ds(col, L)] * t
                         for r in range(RG))

        accs = plsc.parallel_loop(
            0, NCH, 1, unroll=8,
            carry=tuple(jnp.zeros((L,), jnp.float32)
                        for _ in range(RG)))(chunk_body)
        for r in range(RG):
            plsc.store_scatter(den_v,
                               [jnp.full((L,), p0 + r, jnp.int32)],
                               _lane_sum(accs[r]),
                               mask=lane0)

    def trio_body(k, carry):
        for b in range(3):
            wait(b)
            compute_and_store((3 * k + b) * RG, b)

            @pl.when(3 * k + b + 3 < NG)
            def _():
                issue(3 * k + b + 3, b)

        return carry

    lax.fori_loop(0, NG // 3, trio_body, 0)

    pltpu.sync_copy(den_v, part_hbm.at[pl.ds(wid * PD, PD)])
    gather.wait()
    pltpu.sync_copy(pref_v, pref_hbm.at[pl.ds(bout, BPW)])


def _tcmat_body(mat_ref, tblb_ref, strip_ref, tblc_ref, out_ref):
    j = pl.program_id(0)
    col = lax.broadcasted_iota(jnp.int32, (1, CB), 1) + (NB0 + j) * CB
    blk = jnp.where(col < N_SONGS, mat_ref[...], 0.0)
    colv = lax.broadcasted_iota(jnp.int32, (CB, 1), 0) + (NB0 + j) * CB
    tb = jnp.where(colv < N_SONGS, tblb_ref[...], 0.0)
    partial = jnp.dot(blk, tb,
                      preferred_element_type=jnp.float32)

    @pl.when(j == 0)
    def _():
        spart = jnp.dot(strip_ref[...], tblc_ref[...],
                        preferred_element_type=jnp.float32)
        out_ref[...] = partial + jnp.concatenate(
            [jnp.zeros((NG * RG, 1), jnp.float32), spart], axis=0)

    @pl.when(j != 0)
    def _():
        out_ref[...] = out_ref[...] + partial


_tcmat = pl.pallas_call(
    _tcmat_body,
    grid=(NBLK,),
    in_specs=[
        pl.BlockSpec((N_PATTERNS, CB), lambda j: (0, NB0 + j)),
        pl.BlockSpec((CB, 1), lambda j: (NB0 + j, 0)),
        pl.BlockSpec((NROW_TAIL, C0), lambda j: (0, 0)),
        pl.BlockSpec((C0, 1), lambda j: (0, 0)),
    ],
    out_specs=pl.BlockSpec((N_PATTERNS, 1), lambda j: (0, 0)),
    out_shape=jax.ShapeDtypeStruct((N_PATTERNS, 1), jnp.float32),
)


def _combine_body(part_ref, dtc_ref, pref_ref, pat_ref, out_ref):
    den = jnp.sum(part_ref[...], axis=0)
    den = den + jnp.concatenate(
        [dtc_ref[...][:, 0], jnp.zeros((PD - N_PATTERNS,), jnp.float32)])
    oh = (pat_ref[...][:, None] ==
          lax.broadcasted_iota(jnp.int32, (BB, PD), 1)).astype(jnp.float32)
    denb = jnp.dot(oh, den[:, None],
                   preferred_element_type=jnp.float32)
    out_ref[...] = pref_ref[...] / denb[:, 0]


_combine = pl.pallas_call(
    _combine_body,
    grid=(BATCH // BB,),
    in_specs=[
        pl.BlockSpec((NW, PD), lambda j: (0, 0)),
        pl.BlockSpec((N_PATTERNS, 1), lambda j: (0, 0)),
        pl.BlockSpec((BB,), lambda j: (j,)),
        pl.BlockSpec((BB,), lambda j: (j,)),
    ],
    out_specs=pl.BlockSpec((BB,), lambda j: (j,)),
    out_shape=jax.ShapeDtypeStruct((BATCH,), jnp.float32),
)


@jax.jit
def _run(title, pattern, table, mat):
    tbl = table.reshape(-1)
    part, prefs = _scpart(mat, tbl, title)
    dtc = _tcmat(mat, table, mat[NG * RG:, :C0], table)
    out = _combine(part.reshape(NW, PD), dtc, prefs, pattern)
    return out.reshape(-1, 1)


def kernel(title, pattern, table, mat):
    return _run(title, pattern, table, mat)


# speedup vs baseline: 1.9073x; 1.9073x over previous
import functools
import itertools

import jax
import jax.numpy as jnp
import numpy as np
from jax import lax
from jax.experimental import pallas as pl
from jax.experimental.pallas import tpu as pltpu
from jax.experimental.pallas import tpu_sc as plsc

N_SONGS = 100000
N_PATTERNS = 100
BATCH = 16384

NC, NS, L = 2, 16, 16
NW = NC * NS

NG = 12
RG = 8
NROW_TAIL = N_PATTERNS - NG * RG
NT_FULL = N_SONGS // 128
COL_MAIN = NT_FULL * 128
NCOL_TAIL = N_SONGS - COL_MAIN
T_COMMON = NT_FULL // NW
NEXTRA = NT_FULL - T_COMMON * NW
W_COMMON = T_COMMON * 128
W_EXTRA = 128
W = W_COMMON + W_EXTRA
NCH = W // L
NCH_COMMON = W_COMMON // L
PD = 128
BPW = BATCH // NW

_mesh = plsc.VectorSubcoreMesh(core_axis_name="c", subcore_axis_name="s",
                               num_cores=NC, num_subcores=NS)

_GATHER_DNUMS = lax.GatherDimensionNumbers(
    offset_dims=(), collapsed_slice_dims=(0,), start_index_map=(0,))


def _permute(v, perm):
    return lax.gather(v, perm[:, None], _GATHER_DNUMS, slice_sizes=(1,),
                      mode=lax.GatherScatterMode.PROMISE_IN_BOUNDS)


def _lane_sum(v):
    idx = lax.iota(jnp.int32, L)
    for sh in (8, 4, 2, 1):
        v = v + _permute(v, jnp.bitwise_xor(idx, sh))
    return v


@functools.partial(
    pl.kernel,
    out_type=(
        jax.ShapeDtypeStruct((BATCH,), jnp.float32),
        jax.ShapeDtypeStruct((NW * PD,), jnp.float32),
        jax.ShapeDtypeStruct((NC, L), jnp.int32),
    ),
    mesh=_mesh,
    compiler_params=pltpu.CompilerParams(needs_layout_passes=False,
                                         skip_device_barrier=True),
    scratch_types=[
        pltpu.VMEM((W,), jnp.float32),
        pltpu.VMEM((3 * RG, W), jnp.float32),
        pltpu.VMEM((NROW_TAIL * W,), jnp.float32),
        pltpu.VMEM((N_PATTERNS * NCOL_TAIL,), jnp.float32),
        pltpu.VMEM((NCOL_TAIL,), jnp.float32),
        pltpu.VMEM((PD,), jnp.float32),
        pltpu.VMEM((L,), jnp.int32),
        pltpu.VMEM((L,), jnp.int32),
        pltpu.VMEM((NW * PD,), jnp.float32),
        pltpu.VMEM((BPW,), jnp.int32),
        pltpu.VMEM((BPW,), jnp.float32),
        pltpu.VMEM((BPW,), jnp.int32),
        pltpu.VMEM((BPW,), jnp.float32),
        pltpu.SemaphoreType.DMA,
        pltpu.SemaphoreType.DMA,
        pltpu.SemaphoreType.DMA,
        pltpu.SemaphoreType.DMA,
        pltpu.SemaphoreType.DMA,
        pltpu.SemaphoreType.DMA,
    ],
)
def _fused(mat_hbm, tbl_hbm, tail_hbm, ctail_hbm, title_hbm, pattern_hbm,
           token_hbm, out_hbm, part_hbm, flag_hbm,
           tbl_v, buf_v, tail_v, ctail_v, tblr_v, den_v, tok_v, tmp_v,
           part_v, idx_v, pref_v, pat_v, out_v, sem0, sem1, sem2, semt,
           semc, gsem):
    cid = lax.axis_index("c")
    wid = lax.axis_index("s") * NC + cid
    zeros = jnp.zeros((L,), jnp.float32)
    sems = (sem0, sem1, sem2)
    has_extra = wid < NEXTRA
    base = 128 * (T_COMMON * wid + jnp.minimum(wid, NEXTRA))

    def issue(g, b):
        pltpu.async_copy(
            mat_hbm.at[pl.ds(g * RG, RG), pl.ds(base, W_COMMON)],
            buf_v.at[pl.ds(b * RG, RG), pl.ds(0, W_COMMON)], sems[b])

        @pl.when(has_extra)
        def _():
            pltpu.async_copy(
                mat_hbm.at[pl.ds(g * RG, RG), pl.ds(base + W_COMMON,
                                                    W_EXTRA)],
                buf_v.at[pl.ds(b * RG, RG), pl.ds(W_COMMON, W_EXTRA)],
                sems[b])

    def wait(b):
        pltpu.make_async_copy(
            mat_hbm.at[pl.ds(0, RG), pl.ds(0, W_COMMON)],
            buf_v.at[pl.ds(b * RG, RG), pl.ds(0, W_COMMON)], sems[b]).wait()

        @pl.when(has_extra)
        def _():
            pltpu.make_async_copy(
                mat_hbm.at[pl.ds(0, RG), pl.ds(0, W_EXTRA)],
                buf_v.at[pl.ds(b * RG, RG), pl.ds(W_COMMON, W_EXTRA)],
                sems[b]).wait()

    issue(0, 0)
    issue(1, 1)
    issue(2, 2)
    pltpu.sync_copy(tbl_hbm.at[pl.ds(base, W_COMMON)],
                    tbl_v.at[pl.ds(0, W_COMMON)])

    @pl.when(has_extra)
    def _():
        pltpu.sync_copy(tbl_hbm.at[pl.ds(base + W_COMMON, W_EXTRA)],
                        tbl_v.at[pl.ds(W_COMMON, W_EXTRA)])

    for r in range(NROW_TAIL):
        pltpu.async_copy(tail_hbm.at[pl.ds(r * N_SONGS + base, W_COMMON)],
                         tail_v.at[pl.ds(r * W, W_COMMON)], semt)

    @pl.when(has_extra)
    def _():
        for r in range(NROW_TAIL):
            pltpu.async_copy(
                tail_hbm.at[pl.ds(r * N_SONGS + base + W_COMMON, W_EXTRA)],
                tail_v.at[pl.ds(r * W + W_COMMON, W_EXTRA)], semt)

    @pl.when(wid == NW - 1)
    def _():
        pltpu.async_copy(ctail_hbm, ctail_v.at[pl.ds(0, (NG * RG) *
                                                     NCOL_TAIL)], semc)
        for r in range(NROW_TAIL):
            pltpu.async_copy(
                tail_hbm.at[pl.ds(r * N_SONGS + COL_MAIN, NCOL_TAIL)],
                ctail_v.at[pl.ds((NG * RG + r) * NCOL_TAIL, NCOL_TAIL)],
                semc)
        pltpu.async_copy(tbl_hbm.at[pl.ds(COL_MAIN, NCOL_TAIL)], tblr_v,
                         semc)

    bout = wid * BPW
    pltpu.sync_copy(title_hbm.at[pl.ds(bout, BPW)], idx_v)
    gather = pltpu.async_copy(tbl_hbm.at[idx_v], pref_v, gsem)
    pltpu.sync_copy(pattern_hbm.at[pl.ds(bout, BPW)], pat_v)
    pltpu.sync_copy(token_hbm, tok_v)

    for c in range(PD // L):
        den_v[pl.ds(c * L, L)] = zeros

    @pl.when(jnp.logical_not(has_extra))
    def _():
        for c in range(NCH_COMMON, NCH):
            tbl_v[pl.ds(c * L, L)] = zeros
            for r in range(3 * RG):
                buf_v[r, pl.ds(c * L, L)] = zeros
            for r in range(NROW_TAIL):
                tail_v[pl.ds(r * W + c * L, L)] = zeros

    lane0 = lax.iota(jnp.int32, L) == 0

    def compute_and_store(p0, b):
        def chunk_body(j, accs):
            col = j * L
            t = tbl_v[pl.ds(col, L)]
            return tuple(accs[r] + buf_v[b * RG + r, pl.ds(col, L)] * t
                         for r in range(RG))

        accs = plsc.parallel_loop(
            0, NCH, 1, unroll=8,
            carry=tuple(jnp.zeros((L,), jnp.float32)
                        for _ in range(RG)))(chunk_body)
        for r in range(RG):
            plsc.store_scatter(den_v,
                               [jnp.full((L,), p0 + r, jnp.int32)],
                               _lane_sum(accs[r]),
                               mask=lane0)

    def trio_body(k, carry):
        for b in range(3):
            wait(b)
            compute_and_store((3 * k + b) * RG, b)

            @pl.when(3 * k + b + 3 < NG)
            def _():
                issue(3 * k + b + 3, b)

        return carry

    lax.fori_loop(0, NG // 3, trio_body, 0)

    pltpu.make_async_copy(
        tail_hbm.at[pl.ds(0, NROW_TAIL * W_COMMON)],
        tail_v.at[pl.ds(0, NROW_TAIL * W_COMMON)], semt).wait()

    @pl.when(has_extra)
    def _():
        pltpu.make_async_copy(
            tail_hbm.at[pl.ds(0, NROW_TAIL * W_EXTRA)],
            tail_v.at[pl.ds(0, NROW_TAIL * W_EXTRA)], semt).wait()

    @pl.when(wid == NW - 1)
    def _():
        pltpu.make_async_copy(
            tail_hbm.at[pl.ds(0, N_PATTERNS * NCOL_TAIL)],
            ctail_v.at[pl.ds(0, N_PATTERNS * NCOL_TAIL)], semc).wait()
        pltpu.make_async_copy(
            tail_hbm.at[pl.ds(0, NCOL_TAIL)], tblr_v, semc).wait()

    def tail_chunk(j, accs):
        col = j * L
        t = tbl_v[pl.ds(col, L)]
        return tuple(accs[r] + tail_v[pl.ds(r * W + col, L)] * t
                     for r in range(NROW_TAIL))

    taccs = plsc.parallel_loop(
        0, NCH, 1, unroll=4,
        carry=tuple(jnp.zeros((L,), jnp.float32)
                    for _ in range(NROW_TAIL)))(tail_chunk)
    for r in range(NROW_TAIL):
        plsc.store_scatter(den_v,
                           [jnp.full((L,), NG * RG + r, jnp.int32)],
                           _lane_sum(taccs[r]),
                           mask=lane0)

    @pl.when(wid == NW - 1)
    def _():
        def crem_body(p, carry):
            a = (ctail_v[pl.ds(p * NCOL_TAIL, L)] * tblr_v[pl.ds(0, L)] +
                 ctail_v[pl.ds(p * NCOL_TAIL + L, L)] * tblr_v[pl.ds(L, L)])
            pidx = jnp.full((L,), p, jnp.int32)
            cur = plsc.load_gather(den_v, [pidx])
            plsc.store_scatter(den_v, [pidx], cur + _lane_sum(a), mask=lane0)
            return carry

        lax.fori_loop(0, N_PATTERNS, crem_body, 0)

    pltpu.sync_copy(den_v, part_hbm.at[pl.ds(wid * PD, PD)])
    tok = tok_v[pl.ds(0, L)]

    plsc.subcore_barrier()

    @pl.when(lax.axis_index("s") == 0)
    def _():
        pltpu.sync_copy(tok_v, flag_hbm.at[cid])

    def poll_body(done):
        pltpu.sync_copy(flag_hbm.at[1 - cid], tmp_v)
        return jnp.all(tmp_v[pl.ds(0, L)] == tok)

    lax.while_loop(lambda d: jnp.logical_not(d), poll_body,
                   jnp.bool_(False))

    pltpu.sync_copy(part_hbm, part_v)
    for c in range(PD // L):
        acc = jnp.zeros((L,), jnp.float32)
        for w in range(NW):
            acc = acc + part_v[pl.ds(w * PD + c * L, L)]
        den_v[pl.ds(c * L, L)] = acc

    gather.wait()
    for c in range(BPW // L):
        i = pat_v[pl.ds(c * L, L)]
        d = plsc.load_gather(den_v, [i])
        p = pref_v[pl.ds(c * L, L)]
        out_v[pl.ds(c * L, L)] = p / d

    pltpu.sync_copy(out_v, out_hbm.at[pl.ds(bout, BPW)])


_call_counter = itertools.count(1)


@jax.jit
def _run(title, pattern, table, mat, token):
    tbl = table.reshape(-1)
    tail = mat[NG * RG:, :].reshape(-1)
    ctail = mat[:NG * RG, COL_MAIN:].reshape(-1)
    out, _, _ = _fused(mat, tbl, tail, ctail, title, pattern, token)
    return out.reshape(-1, 1)


def kernel(title, pattern, table, mat):
    token = jnp.asarray(
        np.full((L,), (next(_call_counter) % 0x7FFFFFFD) + 1, np.int32))
    return _run(title, pattern, table, mat, token)
